# baseline (speedup 1.0000x reference)
import functools

import jax
import jax.numpy as jnp
from jax import lax
from jax.experimental import pallas as pl
from jax.experimental.pallas import tpu as pltpu
from jax.experimental.pallas import tpu_sc as plsc

N = 10000
E = 320000
EL = 100000
D = 128
H = 128

NPAD = 10240
ELPAD = 102400
NC, NS = 2, 16
NW = NC * NS
EPT = E // NW
EPTP = 10240
ECH = 80
ENCH = EPTP // ECH
HCH = ENCH // 2
DUMP = N
ROWS_PT = NPAD // NS
LPT = ELPAD // NW
LCH = 128
LNCH = LPT // LCH

BM = 2048


def _make_agg():
  mesh = plsc.VectorSubcoreMesh(core_axis_name="c", subcore_axis_name="s")

  @functools.partial(
      pl.kernel,
      out_type=jax.ShapeDtypeStruct((NC, NPAD, D), jnp.float32),
      mesh=mesh,
      scratch_types=[
          pltpu.VMEM((ENCH, ECH), jnp.int32),
          pltpu.VMEM((ENCH, ECH), jnp.int32),
          pltpu.VMEM((ECH, D), jnp.float32),
          pltpu.VMEM_SHARED((NPAD, D), jnp.float32),
          pltpu.SemaphoreType.DMA,
      ],
  )
  def agg(x_hbm, src_hbm, dst_hbm, zeros_hbm, out_hbm,
          src_v, dst_v, rows_v, acc, sem):
    cid = lax.axis_index("c")
    sid = lax.axis_index("s")
    wid = sid * NC + cid
    r0 = sid * ROWS_PT
    pltpu.sync_copy(zeros_hbm.at[pl.ds(r0, ROWS_PT)],
                    acc.at[pl.ds(r0, ROWS_PT)])
    pltpu.sync_copy(src_hbm.at[wid], src_v)
    pltpu.sync_copy(dst_hbm.at[wid], dst_v)
    plsc.subcore_barrier()

    def chunk(k, carry):
      pltpu.async_copy(x_hbm.at[src_v.at[k]], rows_v, sem).wait()
      pltpu.sync_copy(rows_v, acc.at[dst_v.at[k]], add=True)
      return carry

    lax.fori_loop(0, ENCH, chunk, 0)
    plsc.subcore_barrier()
    pltpu.sync_copy(acc.at[pl.ds(r0, ROWS_PT)],
                    out_hbm.at[cid, pl.ds(r0, ROWS_PT)])

  return agg


def _make_deg():
  mesh = plsc.VectorSubcoreMesh(core_axis_name="c", subcore_axis_name="s")

  @functools.partial(
      pl.kernel,
      out_type=jax.ShapeDtypeStruct((NC, NPAD, D), jnp.float32),
      mesh=mesh,
      scratch_types=[
          pltpu.VMEM((ENCH, ECH), jnp.int32),
          pltpu.VMEM((ECH, D), jnp.float32),
          pltpu.VMEM_SHARED((NPAD, D), jnp.float32),
      ],
  )
  def deg(dst_hbm, ones_hbm, zeros_hbm, out_hbm, dst_v, ones_v, acc):
    cid = lax.axis_index("c")
    sid = lax.axis_index("s")
    wid = sid * NC + cid
    r0 = sid * ROWS_PT
    pltpu.sync_copy(zeros_hbm.at[pl.ds(r0, ROWS_PT)],
                    acc.at[pl.ds(r0, ROWS_PT)])
    pltpu.sync_copy(dst_hbm.at[wid], dst_v)
    pltpu.sync_copy(ones_hbm, ones_v)
    plsc.subcore_barrier()

    def chunk(k, carry):
      pltpu.sync_copy(ones_v, acc.at[dst_v.at[k]], add=True)
      return carry

    lax.fori_loop(0, ENCH, chunk, 0)
    plsc.subcore_barrier()
    pltpu.sync_copy(acc.at[pl.ds(r0, ROWS_PT)],
                    out_hbm.at[cid, pl.ds(r0, ROWS_PT)])

  return deg


def _make_dec():
  mesh = plsc.VectorSubcoreMesh(core_axis_name="c", subcore_axis_name="s")

  RPC = LCH // 8

  @functools.partial(
      pl.kernel,
      out_type=jax.ShapeDtypeStruct((ELPAD // 8, D), jnp.float32),
      mesh=mesh,
      scratch_types=[
          pltpu.VMEM((LNCH, LCH), jnp.int32),
          pltpu.VMEM((LNCH, LCH), jnp.int32),
          pltpu.VMEM((LCH, D), jnp.float32),
          pltpu.VMEM((LCH, D), jnp.float32),
          pltpu.VMEM((LCH, D), jnp.float32),
          pltpu.VMEM((LCH, D), jnp.float32),
          pltpu.VMEM((RPC, D), jnp.float32),
          pltpu.VMEM((RPC, D), jnp.float32),
          pltpu.VMEM((8, 16), jnp.float32),
          pltpu.VMEM((8, 16), jnp.float32),
          pltpu.SemaphoreType.DMA,
          pltpu.SemaphoreType.DMA,
          pltpu.SemaphoreType.DMA,
          pltpu.SemaphoreType.DMA,
      ],
  )
  def dec(p_hbm, q_hbm, row_hbm, col_hbm, b1_hbm, w2_hbm, out_hbm,
          row_v, col_v, a0, c0, a1, c1, s0, s1, bias_v, w2_v,
          semG0, semG1, semW0, semW1):
    cid = lax.axis_index("c")
    sid = lax.axis_index("s")
    wid = sid * NC + cid
    base = wid * (LPT // 8)
    pltpu.sync_copy(row_hbm.at[wid], row_v)
    pltpu.sync_copy(col_hbm.at[wid], col_v)
    pltpu.sync_copy(b1_hbm, bias_v)
    pltpu.sync_copy(w2_hbm, w2_v)

    bias = [bias_v[j] for j in range(8)]
    w2 = [w2_v[j] for j in range(8)]

    def compute(a_v, b_v, s_ref):
      def rowbody(r, carry):
        acc = None
        for j in range(8):
          v = (a_v[r, pl.ds(16 * j, 16)] + b_v[r, pl.ds(16 * j, 16)]
               + bias[j])
          v = jnp.maximum(v, 0.0)
          acc = v * w2[j] if acc is None else acc + v * w2[j]
        s_ref[r // 8, pl.ds(16 * (r % 8), 16)] = acc
        return carry

      lax.fori_loop(0, LCH, rowbody, 0)

    sets = [(a0, c0, s0, semG0, semW0), (a1, c1, s1, semG1, semW1)]
    gd = {}
    wd = {}
    av, bv, sv, sg, sw = sets[0]
    gd[0] = (pltpu.async_copy(p_hbm.at[row_v.at[0]], av, sg),
             pltpu.async_copy(q_hbm.at[col_v.at[0]], bv, sg))
    for k in range(LNCH):
      av, bv, sv, sg, sw = sets[k % 2]
      nav, nbv, _, nsg, _ = sets[(k + 1) % 2]
      ga, gb = gd[k]
      ga.wait()
      gb.wait()
      if k + 1 < LNCH:
        gd[k + 1] = (pltpu.async_copy(p_hbm.at[row_v.at[k + 1]], nav, nsg),
                     pltpu.async_copy(q_hbm.at[col_v.at[k + 1]], nbv, nsg))
      if k - 2 >= 0:
        wd[k - 2].wait()
      compute(av, bv, sv)
      wd[k] = pltpu.async_copy(
          sv, out_hbm.at[pl.ds(base + k * RPC, RPC)], sw)
    wd[LNCH - 2].wait()
    wd[LNCH - 1].wait()

  return dec


def _layer1_body(p0, p1, d0, d1, x, wl, wr, bl, out, inv_out):
  s = p0[...][0] + p1[...][0]
  deg = d0[...][0][:, :1] + d1[...][0][:, :1]
  inv = 1.0 / jnp.maximum(deg, 1.0)
  mean = s * inv
  h = jnp.dot(mean, wl[...], preferred_element_type=jnp.float32)
  h = h + jnp.dot(x[...], wr[...], preferred_element_type=jnp.float32)
  h = h + bl[...]
  out[...] = jnp.maximum(h, 0.0)
  inv_out[...] = inv


def _layer2_body(p0, p1, inv, x, wl, wr, bl, out):
  mean = (p0[...][0] + p1[...][0]) * inv[...]
  h = jnp.dot(mean, wl[...], preferred_element_type=jnp.float32)
  h = h + jnp.dot(x[...], wr[...], preferred_element_type=jnp.float32)
  out[...] = jnp.maximum(h + bl[...], 0.0)


def _layer3_body(p0, p1, inv, x, wl, wr, bl, w1a, w1b, pout, qout):
  mean = (p0[...][0] + p1[...][0]) * inv[...]
  z = jnp.dot(mean, wl[...], preferred_element_type=jnp.float32)
  z = z + jnp.dot(x[...], wr[...], preferred_element_type=jnp.float32)
  z = z + bl[...]
  pout[...] = jnp.dot(z, w1a[...], preferred_element_type=jnp.float32)
  qout[...] = jnp.dot(z, w1b[...], preferred_element_type=jnp.float32)


def _dec_tc_body(s, g, b2, out):
  out[...] = (jnp.dot(s[...], g[...], preferred_element_type=jnp.float32)
              + b2[...])


def _row_spec(i_dim, w):
  return pl.BlockSpec((1, BM, w), lambda i, _d=i_dim: (_d, i, 0))


_W128 = pl.BlockSpec((D, D), lambda i: (0, 0))
_B1 = pl.BlockSpec((1, D), lambda i: (0, 0))
_ROW = pl.BlockSpec((BM, D), lambda i: (i, 0))
_COL1 = pl.BlockSpec((BM, 1), lambda i: (i, 0))


def kernel(x_term, edge_index, edge_label_index, Wl1, bl1, Wr1, Wl2, bl2,
           Wr2, Wl3, bl3, Wr3, W1, b1, W2, b2):
  f32 = jnp.float32
  pad_src = jnp.zeros((NW, EPTP - EPT), jnp.int32)
  pad_dst = jnp.full((NW, EPTP - EPT), DUMP, jnp.int32)
  src = (jnp.concatenate([edge_index[0].reshape(NW, EPT), pad_src], axis=1)
         .reshape(NW, ENCH, ECH))
  dst = (jnp.concatenate([edge_index[1].reshape(NW, EPT), pad_dst], axis=1)
         .reshape(NW, ENCH, ECH))

  x_pad = jnp.zeros((NPAD, D), f32).at[:N].set(x_term)
  z128 = jnp.zeros((NPAD, D), f32)
  ones128 = jnp.ones((ECH, D), f32)

  agg128 = _make_agg()

  bl1r = bl1.reshape(1, D)
  bl2r = bl2.reshape(1, D)
  bl3r = bl3.reshape(1, D)
  b1g = b1.reshape(8, 16)
  w2g = W2.reshape(8, 16)
  b2r = b2.reshape(1, 1)
  W1a = W1[:D]
  W1b = W1[D:]

  g = NPAD // BM

  degp = _make_deg()(dst, ones128, z128)
  parts1 = agg128(x_pad, src, dst, z128)
  z1, inv = pl.pallas_call(
      _layer1_body,
      grid=(g,),
      in_specs=[_row_spec(0, D), _row_spec(1, D), _row_spec(0, D),
                _row_spec(1, D), _ROW, _W128, _W128, _B1],
      out_specs=(_ROW, _COL1),
      out_shape=(jax.ShapeDtypeStruct((NPAD, D), f32),
                 jax.ShapeDtypeStruct((NPAD, 1), f32)),
  )(parts1, parts1, degp, degp, x_pad, Wl1, Wr1, bl1r)

  parts2 = agg128(z1, src, dst, z128)
  z2 = pl.pallas_call(
      _layer2_body,
      grid=(g,),
      in_specs=[_row_spec(0, D), _row_spec(1, D), _COL1, _ROW,
                _W128, _W128, _B1],
      out_specs=_ROW,
      out_shape=jax.ShapeDtypeStruct((NPAD, D), f32),
  )(parts2, parts2, inv, z1, Wl2, Wr2, bl2r)

  parts3 = agg128(z2, src, dst, z128)
  P, Q = pl.pallas_call(
      _layer3_body,
      grid=(g,),
      in_specs=[_row_spec(0, D), _row_spec(1, D), _COL1, _ROW,
                _W128, _W128, _B1, _W128, _W128],
      out_specs=(_ROW, _ROW),
      out_shape=(jax.ShapeDtypeStruct((NPAD, D), f32),
                 jax.ShapeDtypeStruct((NPAD, D), f32)),
  )(parts3, parts3, inv, z2, Wl3, Wr3, bl3r, W1a, W1b)

  row = (jnp.zeros((ELPAD,), jnp.int32).at[:EL].set(edge_label_index[0])
         .reshape(NW, LNCH, LCH))
  col = (jnp.zeros((ELPAD,), jnp.int32).at[:EL].set(edge_label_index[1])
         .reshape(NW, LNCH, LCH))
  s8 = _make_dec()(P, Q, row, col, b1g, w2g)

  gmat = (jnp.arange(D)[:, None] // 16 ==
          jnp.arange(8)[None, :]).astype(f32)
  BM8 = 1280
  g8 = (ELPAD // 8) // BM8
  res = pl.pallas_call(
      _dec_tc_body,
      grid=(g8,),
      in_specs=[pl.BlockSpec((BM8, D), lambda i: (i, 0)),
                pl.BlockSpec((D, 8), lambda i: (0, 0)),
                pl.BlockSpec((1, 1), lambda i: (0, 0))],
      out_specs=pl.BlockSpec((BM8, 8), lambda i: (i, 0)),
      out_shape=jax.ShapeDtypeStruct((ELPAD // 8, 8), f32),
  )(s8, gmat, b2r)
  return res.reshape(ELPAD)[:EL]


# speedup vs baseline: 1.8440x; 1.8440x over previous
import functools

import jax
import jax.numpy as jnp
from jax import lax
from jax.experimental import pallas as pl
from jax.experimental.pallas import tpu as pltpu
from jax.experimental.pallas import tpu_sc as plsc

N = 10000
E = 320000
EL = 100000
D = 128
H = 128

NPAD = 10240
ELPAD = 102400
NC, NS = 2, 16
NW = NC * NS
EPT = E // NW
ECH = 80
ENCH = EPT // ECH
ROWS_PT = NPAD // NS
LPT = ELPAD // NW
LCH = 128
LNCH = LPT // LCH

BM = 2048


def _make_agg():
  mesh = plsc.VectorSubcoreMesh(core_axis_name="c", subcore_axis_name="s")

  @functools.partial(
      pl.kernel,
      out_type=jax.ShapeDtypeStruct((NC, NPAD, D), jnp.float32),
      mesh=mesh,
      scratch_types=[
          pltpu.VMEM((ENCH, ECH), jnp.int32),
          pltpu.VMEM((ENCH, ECH), jnp.int32),
          pltpu.VMEM((ECH, D), jnp.float32),
          pltpu.VMEM_SHARED((NPAD, D), jnp.float32),
          pltpu.SemaphoreType.DMA,
      ],
  )
  def agg(x_hbm, src_hbm, dst_hbm, zeros_hbm, out_hbm,
          src_v, dst_v, rows_v, acc, sem):
    cid = lax.axis_index("c")
    sid = lax.axis_index("s")
    wid = sid * NC + cid
    r0 = sid * ROWS_PT
    pltpu.sync_copy(zeros_hbm.at[pl.ds(r0, ROWS_PT)],
                    acc.at[pl.ds(r0, ROWS_PT)])
    pltpu.sync_copy(src_hbm.at[wid], src_v)
    pltpu.sync_copy(dst_hbm.at[wid], dst_v)
    plsc.subcore_barrier()

    def chunk(k, carry):
      pltpu.async_copy(x_hbm.at[src_v.at[k]], rows_v, sem).wait()
      pltpu.sync_copy(rows_v, acc.at[dst_v.at[k]], add=True)
      return carry

    lax.fori_loop(0, ENCH, chunk, 0)
    plsc.subcore_barrier()
    pltpu.sync_copy(acc.at[pl.ds(r0, ROWS_PT)],
                    out_hbm.at[cid, pl.ds(r0, ROWS_PT)])

  return agg


def _make_deg():
  mesh = plsc.VectorSubcoreMesh(core_axis_name="c", subcore_axis_name="s")

  @functools.partial(
      pl.kernel,
      out_type=jax.ShapeDtypeStruct((NC, NPAD, D), jnp.float32),
      mesh=mesh,
      scratch_types=[
          pltpu.VMEM((ENCH, ECH), jnp.int32),
          pltpu.VMEM((ECH, D), jnp.float32),
          pltpu.VMEM_SHARED((NPAD, D), jnp.float32),
      ],
  )
  def deg(dst_hbm, ones_hbm, zeros_hbm, out_hbm, dst_v, ones_v, acc):
    cid = lax.axis_index("c")
    sid = lax.axis_index("s")
    wid = sid * NC + cid
    r0 = sid * ROWS_PT
    pltpu.sync_copy(zeros_hbm.at[pl.ds(r0, ROWS_PT)],
                    acc.at[pl.ds(r0, ROWS_PT)])
    pltpu.sync_copy(dst_hbm.at[wid], dst_v)
    pltpu.sync_copy(ones_hbm, ones_v)
    plsc.subcore_barrier()

    def chunk(k, carry):
      pltpu.sync_copy(ones_v, acc.at[dst_v.at[k]], add=True)
      return carry

    lax.fori_loop(0, ENCH, chunk, 0)
    plsc.subcore_barrier()
    pltpu.sync_copy(acc.at[pl.ds(r0, ROWS_PT)],
                    out_hbm.at[cid, pl.ds(r0, ROWS_PT)])

  return deg


def _make_dec():
  mesh = plsc.VectorSubcoreMesh(core_axis_name="c", subcore_axis_name="s")

  RPC = LCH // 8

  @functools.partial(
      pl.kernel,
      out_type=jax.ShapeDtypeStruct((ELPAD // 8, D), jnp.float32),
      mesh=mesh,
      scratch_types=[
          pltpu.VMEM((LNCH, LCH), jnp.int32),
          pltpu.VMEM((LNCH, LCH), jnp.int32),
          pltpu.VMEM((LCH, D), jnp.float32),
          pltpu.VMEM((LCH, D), jnp.float32),
          pltpu.VMEM((LCH, D), jnp.float32),
          pltpu.VMEM((LCH, D), jnp.float32),
          pltpu.VMEM((RPC, D), jnp.float32),
          pltpu.VMEM((RPC, D), jnp.float32),
          pltpu.VMEM((8, 16), jnp.float32),
          pltpu.VMEM((8, 16), jnp.float32),
          pltpu.SemaphoreType.DMA,
          pltpu.SemaphoreType.DMA,
          pltpu.SemaphoreType.DMA,
          pltpu.SemaphoreType.DMA,
      ],
  )
  def dec(p_hbm, q_hbm, row_hbm, col_hbm, b1_hbm, w2_hbm, out_hbm,
          row_v, col_v, a0, c0, a1, c1, s0, s1, bias_v, w2_v,
          semG0, semG1, semW0, semW1):
    cid = lax.axis_index("c")
    sid = lax.axis_index("s")
    wid = sid * NC + cid
    base = wid * (LPT // 8)
    pltpu.sync_copy(row_hbm.at[wid], row_v)
    pltpu.sync_copy(col_hbm.at[wid], col_v)
    pltpu.sync_copy(b1_hbm, bias_v)
    pltpu.sync_copy(w2_hbm, w2_v)

    bias = [bias_v[j] for j in range(8)]
    w2 = [w2_v[j] for j in range(8)]

    def compute(a_v, b_v, s_ref):
      def rowbody(r, carry):
        acc = None
        for j in range(8):
          v = (a_v[r, pl.ds(16 * j, 16)] + b_v[r, pl.ds(16 * j, 16)]
               + bias[j])
          v = jnp.maximum(v, 0.0)
          acc = v * w2[j] if acc is None else acc + v * w2[j]
        s_ref[r // 8, pl.ds(16 * (r % 8), 16)] = acc
        return carry

      lax.fori_loop(0, LCH, rowbody, 0)

    sets = [(a0, c0, s0, semG0, semW0), (a1, c1, s1, semG1, semW1)]
    gd = {}
    wd = {}
    av, bv, sv, sg, sw = sets[0]
    gd[0] = (pltpu.async_copy(p_hbm.at[row_v.at[0]], av, sg),
             pltpu.async_copy(q_hbm.at[col_v.at[0]], bv, sg))
    for k in range(LNCH):
      av, bv, sv, sg, sw = sets[k % 2]
      nav, nbv, _, nsg, _ = sets[(k + 1) % 2]
      ga, gb = gd[k]
      ga.wait()
      gb.wait()
      if k + 1 < LNCH:
        gd[k + 1] = (pltpu.async_copy(p_hbm.at[row_v.at[k + 1]], nav, nsg),
                     pltpu.async_copy(q_hbm.at[col_v.at[k + 1]], nbv, nsg))
      if k - 2 >= 0:
        wd[k - 2].wait()
      compute(av, bv, sv)
      wd[k] = pltpu.async_copy(
          sv, out_hbm.at[pl.ds(base + k * RPC, RPC)], sw)
    wd[LNCH - 2].wait()
    wd[LNCH - 1].wait()

  return dec


def _layer1_body(p0, p1, d0, d1, x, wl, wr, bl, out, inv_out):
  s = p0[...][0] + p1[...][0]
  deg = d0[...][0][:, :1] + d1[...][0][:, :1]
  inv = 1.0 / jnp.maximum(deg, 1.0)
  mean = s * inv
  h = jnp.dot(mean, wl[...], preferred_element_type=jnp.float32)
  h = h + jnp.dot(x[...], wr[...], preferred_element_type=jnp.float32)
  h = h + bl[...]
  out[...] = jnp.maximum(h, 0.0)
  inv_out[...] = inv


def _layer2_body(p0, p1, inv, x, wl, wr, bl, out):
  mean = (p0[...][0] + p1[...][0]) * inv[...]
  h = jnp.dot(mean, wl[...], preferred_element_type=jnp.float32)
  h = h + jnp.dot(x[...], wr[...], preferred_element_type=jnp.float32)
  out[...] = jnp.maximum(h + bl[...], 0.0)


def _layer3_body(p0, p1, inv, x, wl, wr, bl, w1a, w1b, pout, qout):
  mean = (p0[...][0] + p1[...][0]) * inv[...]
  z = jnp.dot(mean, wl[...], preferred_element_type=jnp.float32)
  z = z + jnp.dot(x[...], wr[...], preferred_element_type=jnp.float32)
  z = z + bl[...]
  pout[...] = jnp.dot(z, w1a[...], preferred_element_type=jnp.float32)
  qout[...] = jnp.dot(z, w1b[...], preferred_element_type=jnp.float32)


def _dec_tc_body(s, g, b2, out):
  out[...] = (jnp.dot(s[...], g[...], preferred_element_type=jnp.float32)
              + b2[...])


def _row_spec(i_dim, w):
  return pl.BlockSpec((1, BM, w), lambda i, _d=i_dim: (_d, i, 0))


_W128 = pl.BlockSpec((D, D), lambda i: (0, 0))
_B1 = pl.BlockSpec((1, D), lambda i: (0, 0))
_ROW = pl.BlockSpec((BM, D), lambda i: (i, 0))
_COL1 = pl.BlockSpec((BM, 1), lambda i: (i, 0))


def kernel(x_term, edge_index, edge_label_index, Wl1, bl1, Wr1, Wl2, bl2,
           Wr2, Wl3, bl3, Wr3, W1, b1, W2, b2):
  f32 = jnp.float32
  src = edge_index[0].reshape(NW, ENCH, ECH)
  dst = edge_index[1].reshape(NW, ENCH, ECH)

  x_pad = jnp.zeros((NPAD, D), f32).at[:N].set(x_term)
  z128 = jnp.zeros((NPAD, D), f32)
  ones128 = jnp.ones((ECH, D), f32)

  agg128 = _make_agg()

  bl1r = bl1.reshape(1, D)
  bl2r = bl2.reshape(1, D)
  bl3r = bl3.reshape(1, D)
  b1g = b1.reshape(8, 16)
  w2g = W2.reshape(8, 16)
  b2r = b2.reshape(1, 1)
  W1a = W1[:D]
  W1b = W1[D:]

  g = NPAD // BM

  degp = _make_deg()(dst, ones128, z128)
  parts1 = agg128(x_pad, src, dst, z128)
  z1, inv = pl.pallas_call(
      _layer1_body,
      grid=(g,),
      in_specs=[_row_spec(0, D), _row_spec(1, D), _row_spec(0, D),
                _row_spec(1, D), _ROW, _W128, _W128, _B1],
      out_specs=(_ROW, _COL1),
      out_shape=(jax.ShapeDtypeStruct((NPAD, D), f32),
                 jax.ShapeDtypeStruct((NPAD, 1), f32)),
  )(parts1, parts1, degp, degp, x_pad, Wl1, Wr1, bl1r)

  parts2 = agg128(z1, src, dst, z128)
  z2 = pl.pallas_call(
      _layer2_body,
      grid=(g,),
      in_specs=[_row_spec(0, D), _row_spec(1, D), _COL1, _ROW,
                _W128, _W128, _B1],
      out_specs=_ROW,
      out_shape=jax.ShapeDtypeStruct((NPAD, D), f32),
  )(parts2, parts2, inv, z1, Wl2, Wr2, bl2r)

  parts3 = agg128(z2, src, dst, z128)
  P, Q = pl.pallas_call(
      _layer3_body,
      grid=(g,),
      in_specs=[_row_spec(0, D), _row_spec(1, D), _COL1, _ROW,
                _W128, _W128, _B1, _W128, _W128],
      out_specs=(_ROW, _ROW),
      out_shape=(jax.ShapeDtypeStruct((NPAD, D), f32),
                 jax.ShapeDtypeStruct((NPAD, D), f32)),
  )(parts3, parts3, inv, z2, Wl3, Wr3, bl3r, W1a, W1b)

  row = (jnp.zeros((ELPAD,), jnp.int32).at[:EL].set(edge_label_index[0])
         .reshape(NW, LNCH, LCH))
  col = (jnp.zeros((ELPAD,), jnp.int32).at[:EL].set(edge_label_index[1])
         .reshape(NW, LNCH, LCH))
  s8 = _make_dec()(P, Q, row, col, b1g, w2g)

  gmat = (jnp.arange(D)[:, None] // 16 ==
          jnp.arange(8)[None, :]).astype(f32)
  BM8 = 1280
  g8 = (ELPAD // 8) // BM8
  res = pl.pallas_call(
      _dec_tc_body,
      grid=(g8,),
      in_specs=[pl.BlockSpec((BM8, D), lambda i: (i, 0)),
                pl.BlockSpec((D, 8), lambda i: (0, 0)),
                pl.BlockSpec((1, 1), lambda i: (0, 0))],
      out_specs=pl.BlockSpec((BM8, 8), lambda i: (i, 0)),
      out_shape=jax.ShapeDtypeStruct((ELPAD // 8, 8), f32),
  )(s8, gmat, b2r)
  return res.reshape(ELPAD)[:EL]
